# Initial kernel scaffold; baseline (speedup 1.0000x reference)
import jax
import jax.numpy as jnp
from jax.experimental import pallas as pl


def kernel(x, edge_index, W1, b1, W2, b2):
    raise NotImplementedError("write your pallas kernel here")


# speedup vs baseline: 1.7459x; 1.7459x over previous
import jax
import jax.numpy as jnp
from jax.experimental import pallas as pl

N = 10000


def kernel(x, edge_index, W1, b1, W2, b2):
    src = edge_index[0]
    dst = edge_index[1]
    ones = jnp.ones((src.shape[0],), jnp.float32)
    deg = jax.ops.segment_sum(ones, src, num_segments=N)
    dis = jnp.where(deg > 0, deg ** -0.5, 0.0)

    def S(g):
        return jax.ops.segment_sum(g[src], dst, num_segments=N)

    def layer(h, W, b):
        y0m2 = h @ (W[0] - W[2])
        g1 = dis[:, None] * (h @ W[1])
        g2 = dis[:, None] * (h @ W[2])
        t = S(g2)
        gp = g1 - 2.0 * (dis * dis)[:, None] * t
        u = S(gp)
        return y0m2 - dis[:, None] * u + b

    h = jax.nn.relu(layer(x, W1, b1))
    return layer(h, W2, b2)


# speedup vs baseline: 5.2778x; 3.0230x over previous
import functools

import jax
import jax.numpy as jnp
from jax import lax
from jax.experimental import pallas as pl
from jax.experimental.pallas import tpu as pltpu
from jax.experimental.pallas import tpu_sc as plsc

N = 10000
E = 320000
NP = 10240
F_IN = 128
HID = 64
CLS = 16
FH = HID // 2
HALFN = NP // 2
ACCR = HALFN + 8
DUMMY = HALFN
C = 80
NSUB = 16
ROWS1 = NP // NSUB
ROWS2 = HALFN // NSUB
EPT_E = E // 32
EPT_C = E // 16

_mesh = lambda: plsc.VectorSubcoreMesh(core_axis_name="c", subcore_axis_name="s")
_SC_PARAMS = pltpu.CompilerParams(use_tc_tiling_on_sc=False)
_f32 = jnp.float32
_HIGH = jax.lax.Precision.HIGHEST


def _dot(a, b):
    return jax.lax.dot(a, b, precision=_HIGH, preferred_element_type=_f32)


def _deg_kernel():
    @functools.partial(
        pl.kernel,
        out_type=jax.ShapeDtypeStruct((2, NP, 16), _f32),
        mesh=_mesh(),
        compiler_params=_SC_PARAMS,
        scratch_types=[
            pltpu.VMEM_SHARED((NP, 16), _f32),
            pltpu.VMEM((C,), jnp.int32),
            pltpu.VMEM((C, 16), _f32),
            pltpu.VMEM((ROWS1, 16), _f32),
        ],
    )
    def k(src_hbm, out_hbm, acc, sidx, ones_v, rbuf):
        cid = lax.axis_index("c")
        sid = lax.axis_index("s")
        zero16 = jnp.zeros((16,), _f32)
        pat = jnp.where(lax.iota(jnp.int32, 16) == 0, 1.0, 0.0).astype(_f32)

        def fill(i, _):
            rbuf[i, :] = zero16
            return 0
        lax.fori_loop(0, ROWS1, fill, 0)

        def fill1(i, _):
            ones_v[i, :] = pat
            return 0
        lax.fori_loop(0, C, fill1, 0)

        pltpu.sync_copy(rbuf, acc.at[pl.ds(sid * ROWS1, ROWS1)])
        plsc.subcore_barrier()

        wid = sid * 2 + cid
        base = wid * EPT_E

        def chunk(kk, _):
            off = base + kk * C
            pltpu.sync_copy(src_hbm.at[pl.ds(off, C)], sidx)
            pltpu.sync_copy(ones_v, acc.at[sidx], add=True)
            return 0
        lax.fori_loop(0, EPT_E // C, chunk, 0)

        plsc.subcore_barrier()
        r0 = sid * ROWS1
        pltpu.sync_copy(acc.at[pl.ds(r0, ROWS1)], rbuf)
        pltpu.sync_copy(rbuf, out_hbm.at[cid, pl.ds(r0, ROWS1)])

    return k


def _prop_col_kernel():
    @functools.partial(
        pl.kernel,
        out_type=jax.ShapeDtypeStruct((2, NP, FH), _f32),
        mesh=_mesh(),
        compiler_params=_SC_PARAMS,
        scratch_types=[
            pltpu.VMEM_SHARED((NP, FH), _f32),
            pltpu.VMEM((C,), jnp.int32),
            pltpu.VMEM((C,), jnp.int32),
            pltpu.VMEM((C, FH), _f32),
            pltpu.VMEM((ROWS1, FH), _f32),
            pltpu.VMEM((ROWS1, FH), _f32),
            pltpu.VMEM((ROWS1, FH), _f32),
            pltpu.SemaphoreType.DMA,
        ],
    )
    def k(src_hbm, dst_hbm, tab_hbm, add_hbm, scal_hbm, out_hbm,
          acc, sidx, didx, rows, abuf, obuf, sbuf, sem):
        cid = lax.axis_index("c")
        sid = lax.axis_index("s")
        zero16 = jnp.zeros((16,), _f32)

        def fill(i, _):
            obuf[i, pl.ds(0, 16)] = zero16
            obuf[i, pl.ds(16, 16)] = zero16
            return 0
        lax.fori_loop(0, ROWS1, fill, 0)
        pltpu.sync_copy(obuf, acc.at[pl.ds(sid * ROWS1, ROWS1)])
        plsc.subcore_barrier()

        base = sid * EPT_C
        tab_off = cid * NP

        def chunk(kk, _):
            off = base + kk * C
            pltpu.sync_copy(src_hbm.at[pl.ds(off, C)], sidx)
            pltpu.sync_copy(dst_hbm.at[pl.ds(off, C)], didx)
            for j in range(C // 16):
                s = pl.ds(j * 16, 16)
                sidx[s] = sidx[s] + tab_off
            pltpu.async_copy(tab_hbm.at[sidx], rows, sem).wait()
            pltpu.sync_copy(rows, acc.at[didx], add=True)
            return 0
        lax.fori_loop(0, EPT_C // C, chunk, 0)

        plsc.subcore_barrier()
        r0 = sid * ROWS1
        pltpu.sync_copy(acc.at[pl.ds(r0, ROWS1)], abuf)
        pltpu.sync_copy(add_hbm.at[cid, pl.ds(r0, ROWS1)], obuf)
        pltpu.sync_copy(scal_hbm.at[pl.ds(r0, ROWS1)], sbuf)

        def row(i, _):
            for j in range(FH // 16):
                s = pl.ds(j * 16, 16)
                obuf[i, s] = obuf[i, s] + sbuf[i, s] * abuf[i, s]
            return 0
        lax.fori_loop(0, ROWS1, row, 0)
        pltpu.sync_copy(obuf, out_hbm.at[cid, pl.ds(r0, ROWS1)])

    return k


def _prop_node_kernel(with_bias):
    scratch = [
        pltpu.VMEM_SHARED((ACCR, 16), _f32),
        pltpu.VMEM((C,), jnp.int32),
        pltpu.VMEM((C,), jnp.int32),
        pltpu.VMEM((C, 16), _f32),
        pltpu.VMEM((ROWS2, 16), _f32),
        pltpu.VMEM((ROWS2, 16), _f32),
        pltpu.VMEM((ROWS2, 16), _f32),
        pltpu.VMEM((16,), _f32),
        pltpu.SemaphoreType.DMA,
    ]

    @functools.partial(
        pl.kernel,
        out_type=jax.ShapeDtypeStruct((NP, CLS), _f32),
        mesh=_mesh(),
        compiler_params=_SC_PARAMS,
        scratch_types=scratch,
    )
    def k(src_hbm, dst_hbm, tab_hbm, add_hbm, scal_hbm, bias_hbm, out_hbm,
          acc, sidx, didx, rows, abuf, obuf, sbuf, bbuf, sem):
        cid = lax.axis_index("c")
        sid = lax.axis_index("s")
        zero16 = jnp.zeros((16,), _f32)

        def fillz(i, _):
            obuf[i, :] = zero16
            return 0
        lax.fori_loop(0, ROWS2, fillz, 0)
        pltpu.sync_copy(obuf, acc.at[pl.ds(sid * ROWS2, ROWS2)])

        @pl.when(sid == 0)
        def _():
            pltpu.sync_copy(obuf.at[pl.ds(0, 8)], acc.at[pl.ds(HALFN, 8)])

        plsc.subcore_barrier()

        base = sid * EPT_C
        node0 = cid * HALFN

        def chunk(kk, _):
            off = base + kk * C
            pltpu.sync_copy(src_hbm.at[pl.ds(off, C)], sidx)
            pltpu.sync_copy(dst_hbm.at[pl.ds(off, C)], didx)
            for j in range(C // 16):
                s = pl.ds(j * 16, 16)
                dv = didx[s] - node0
                ok = (dv >= 0) & (dv < HALFN)
                didx[s] = jnp.where(ok, dv, DUMMY)
            pltpu.async_copy(tab_hbm.at[sidx], rows, sem).wait()
            pltpu.sync_copy(rows, acc.at[didx], add=True)
            return 0
        lax.fori_loop(0, EPT_C // C, chunk, 0)

        plsc.subcore_barrier()
        r0l = sid * ROWS2
        r0g = cid * HALFN + r0l
        pltpu.sync_copy(acc.at[pl.ds(r0l, ROWS2)], abuf)
        pltpu.sync_copy(add_hbm.at[pl.ds(r0g, ROWS2)], obuf)
        pltpu.sync_copy(scal_hbm.at[pl.ds(r0g, ROWS2)], sbuf)
        pltpu.sync_copy(bias_hbm, bbuf)
        bvec = bbuf[:] if with_bias else zero16

        def row(i, _):
            obuf[i, :] = obuf[i, :] + sbuf[i, :] * abuf[i, :] + bvec
            return 0
        lax.fori_loop(0, ROWS2, row, 0)
        pltpu.sync_copy(obuf, out_hbm.at[pl.ds(r0g, ROWS2)])

    return k


def _mm1_body(x_ref, degp_ref, w_ref,
              y_ref, g1_ref, g2_ref, dis_ref, sa1_ref, sb1_ref,
              sa2_ref, sb2_ref):
    deg = degp_ref[0, :, 0:1] + degp_ref[1, :, 0:1]
    dis = jnp.where(deg > 0, jax.lax.rsqrt(deg), 0.0)
    dis_ref[...] = dis
    dis2 = dis * dis
    bm = dis.shape[0]
    sa1_ref[...] = jnp.broadcast_to(-2.0 * dis2, (bm, FH))
    sb1_ref[...] = jnp.broadcast_to(-dis, (bm, FH))
    sa2_ref[...] = jnp.broadcast_to(-2.0 * dis2, (bm, CLS))
    sb2_ref[...] = jnp.broadcast_to(-dis, (bm, CLS))
    x = x_ref[...]
    y = _dot(x, w_ref[0] - w_ref[2])
    g1 = dis * _dot(x, w_ref[1])
    g2 = dis * _dot(x, w_ref[2])
    y_ref[0] = y[:, :FH]
    y_ref[1] = y[:, FH:]
    g1_ref[0] = g1[:, :FH]
    g1_ref[1] = g1[:, FH:]
    g2_ref[0] = g2[:, :FH]
    g2_ref[1] = g2[:, FH:]


def _mm1_call(xp, degp, W1):
    BM = 1024
    grid = (NP // BM,)
    outs = [
        jax.ShapeDtypeStruct((2, NP, FH), _f32),
        jax.ShapeDtypeStruct((2, NP, FH), _f32),
        jax.ShapeDtypeStruct((2, NP, FH), _f32),
        jax.ShapeDtypeStruct((NP, 1), _f32),
        jax.ShapeDtypeStruct((NP, FH), _f32),
        jax.ShapeDtypeStruct((NP, FH), _f32),
        jax.ShapeDtypeStruct((NP, CLS), _f32),
        jax.ShapeDtypeStruct((NP, CLS), _f32),
    ]
    in_specs = [
        pl.BlockSpec((BM, F_IN), lambda i: (i, 0)),
        pl.BlockSpec((2, BM, 16), lambda i: (0, i, 0)),
        pl.BlockSpec((3, F_IN, HID), lambda i: (0, 0, 0)),
    ]
    out_specs = [
        pl.BlockSpec((2, BM, FH), lambda i: (0, i, 0)),
        pl.BlockSpec((2, BM, FH), lambda i: (0, i, 0)),
        pl.BlockSpec((2, BM, FH), lambda i: (0, i, 0)),
        pl.BlockSpec((BM, 1), lambda i: (i, 0)),
        pl.BlockSpec((BM, FH), lambda i: (i, 0)),
        pl.BlockSpec((BM, FH), lambda i: (i, 0)),
        pl.BlockSpec((BM, CLS), lambda i: (i, 0)),
        pl.BlockSpec((BM, CLS), lambda i: (i, 0)),
    ]
    return pl.pallas_call(_mm1_body, grid=grid, in_specs=in_specs,
                          out_specs=out_specs, out_shape=outs)(xp, degp, W1)


def _mm2_body(z_ref, w_ref, b1_ref, dis_ref, v_ref, gg1_ref, gg2_ref):
    z = jnp.concatenate([z_ref[0], z_ref[1]], axis=1)
    h = jnp.maximum(z + b1_ref[...], 0.0)
    dis = dis_ref[...]
    v_ref[...] = _dot(h, w_ref[0] - w_ref[2])
    gg1_ref[...] = dis * _dot(h, w_ref[1])
    gg2_ref[...] = dis * _dot(h, w_ref[2])


def _mm2_call(zs, W2, b1, dis):
    BM = 1024
    grid = (NP // BM,)
    outs = [
        jax.ShapeDtypeStruct((NP, CLS), _f32),
        jax.ShapeDtypeStruct((NP, CLS), _f32),
        jax.ShapeDtypeStruct((NP, CLS), _f32),
    ]
    in_specs = [
        pl.BlockSpec((2, BM, FH), lambda i: (0, i, 0)),
        pl.BlockSpec((3, HID, CLS), lambda i: (0, 0, 0)),
        pl.BlockSpec((1, HID), lambda i: (0, 0)),
        pl.BlockSpec((BM, 1), lambda i: (i, 0)),
    ]
    out_specs = [
        pl.BlockSpec((BM, CLS), lambda i: (i, 0)),
        pl.BlockSpec((BM, CLS), lambda i: (i, 0)),
        pl.BlockSpec((BM, CLS), lambda i: (i, 0)),
    ]
    return pl.pallas_call(_mm2_body, grid=grid, in_specs=in_specs,
                          out_specs=out_specs,
                          out_shape=outs)(zs, W2, b1.reshape(1, HID), dis)


_K_DEG = _deg_kernel()
_K_S1 = _prop_col_kernel()
_K_S2 = _prop_col_kernel()
_K_S3 = _prop_node_kernel(with_bias=False)
_K_S4 = _prop_node_kernel(with_bias=True)


def kernel(x, edge_index, W1, b1, W2, b2):
    xp = jnp.pad(x, ((0, NP - N), (0, 0)))
    src = edge_index[0]
    dst = edge_index[1]
    degp = _K_DEG(src)
    ys, g1s, g2s, dis, sa1, sb1, sa2, sb2 = _mm1_call(xp, degp, W1)

    g2f = g2s.reshape(2 * NP, FH)
    gps = _K_S1(src, dst, g2f, g1s, sa1)
    gpf = gps.reshape(2 * NP, FH)
    zs = _K_S2(src, dst, gpf, ys, sb1)

    v0m2, gg1, gg2 = _mm2_call(zs, W2, b1, dis)
    gp2 = _K_S3(src, dst, gg2, gg1, sa2, b2)
    outp = _K_S4(src, dst, gp2, v0m2, sb2, b2)
    return outp[:N]


# speedup vs baseline: 20.2221x; 3.8316x over previous
import functools

import jax
import jax.numpy as jnp
from jax import lax
from jax.experimental import pallas as pl
from jax.experimental.pallas import tpu as pltpu
from jax.experimental.pallas import tpu_sc as plsc

N = 10000
E = 320000
NP = 10240
F_IN = 128
HID = 64
CLS = 16
FH = HID // 2
HALFN = NP // 2
ACCR = HALFN + 8
DUMMY = HALFN
C = 128
EPAD = 327680
NSUB = 16
ROWS1 = NP // NSUB
ROWS2 = HALFN // NSUB
NCHK_E = EPAD // 32 // C
NCHK_C = EPAD // 16 // C
NB_D = 16
NRING = 8
NCHK_H = NCHK_C // 2
NGRP = NCHK_H // NRING
NCHK_Q = NCHK_C // 4
NGRP_Q = NCHK_Q // NRING
EROWS = ROWS1 // 2
QROWS = ROWS1 // 4

_mesh = lambda: plsc.VectorSubcoreMesh(core_axis_name="c", subcore_axis_name="s")
_SC_PARAMS = pltpu.CompilerParams(use_tc_tiling_on_sc=False)
_f32 = jnp.float32
_HIGH = jax.lax.Precision.HIGHEST


def _dot(a, b):
    return jax.lax.dot(a, b, precision=_HIGH, preferred_element_type=_f32)


def _deg_kernel():
    @functools.partial(
        pl.kernel,
        out_type=jax.ShapeDtypeStruct((2, NP, 16), _f32),
        mesh=_mesh(),
        compiler_params=_SC_PARAMS,
        scratch_types=[
            pltpu.VMEM_SHARED((NP, 16), _f32),
            pltpu.VMEM((NB_D, C), jnp.int32),
            pltpu.VMEM((C, 16), _f32),
            pltpu.VMEM((ROWS1, 16), _f32),
            pltpu.SemaphoreType.DMA,
        ],
    )
    def k(src2_hbm, out_hbm, acc, sbuf2, ones_v, rbuf, ssem):
        cid = lax.axis_index("c")
        sid = lax.axis_index("s")
        zero16 = jnp.zeros((16,), _f32)
        pat = jnp.where(lax.iota(jnp.int32, 16) == 0, 1.0, 0.0).astype(_f32)

        def fill(i, _):
            rbuf[i, :] = zero16
            return 0
        lax.fori_loop(0, ROWS1, fill, 0)

        def fill1(i, _):
            ones_v[i, :] = pat
            return 0
        lax.fori_loop(0, C, fill1, 0)

        pltpu.sync_copy(rbuf, acc.at[pl.ds(sid * ROWS1, ROWS1)])
        plsc.subcore_barrier()

        wid = sid * 2 + cid
        rbase = wid * NCHK_E

        def blk(b, _):
            pltpu.sync_copy(src2_hbm.at[pl.ds(rbase + b * NB_D, NB_D)], sbuf2)

            def fire(j, _):
                pltpu.async_copy(ones_v, acc.at[sbuf2.at[j]], ssem, add=True)
                return 0
            lax.fori_loop(0, NB_D, fire, 0)

            def drain(j, _):
                pltpu.make_async_copy(out_hbm.at[0, pl.ds(0, C)], ones_v, ssem).wait()
                return 0
            lax.fori_loop(0, NB_D, drain, 0)
            return 0
        lax.fori_loop(0, NCHK_E // NB_D, blk, 0)

        plsc.subcore_barrier()
        r0 = sid * ROWS1
        pltpu.sync_copy(acc.at[pl.ds(r0, ROWS1)], rbuf)
        pltpu.sync_copy(rbuf, out_hbm.at[cid, pl.ds(r0, ROWS1)])

    return k


def _layer1_kernel():
    @functools.partial(
        pl.kernel,
        out_type=jax.ShapeDtypeStruct((2, NP, FH), _f32),
        mesh=_mesh(),
        compiler_params=_SC_PARAMS,
        scratch_types=[
            pltpu.VMEM_SHARED((NP, FH), _f32),
            pltpu.VMEM_SHARED((NP, FH), _f32),
            pltpu.VMEM_SHARED((NP, FH), _f32),
            pltpu.VMEM((NCHK_Q, C), jnp.int32),
            pltpu.VMEM((NCHK_Q, C), jnp.int32),
            [pltpu.VMEM((C, FH), _f32)] * NRING,
            pltpu.VMEM((QROWS, FH), _f32),
            pltpu.VMEM((QROWS, FH), _f32),
            pltpu.VMEM((QROWS, FH), _f32),
            [pltpu.SemaphoreType.DMA] * NRING,
            [pltpu.SemaphoreType.DMA] * NRING,
        ],
    )
    def k(src2_hbm, dst2_hbm, tab_hbm, g1_hbm, ys_hbm, sa1_hbm, sb1_hbm, out_hbm,
          tbl_s, acc_a, acc_b, sbuf2, dbuf2, ring, abuf, obuf, sbuf, gsem, ssem):
        cid = lax.axis_index("c")
        sid = lax.axis_index("s")
        zero16 = jnp.zeros((16,), _f32)
        rbase = sid * NCHK_C
        tab_off = cid * NP

        def fill(i, _):
            obuf[i, pl.ds(0, 16)] = zero16
            obuf[i, pl.ds(16, 16)] = zero16
            return 0
        lax.fori_loop(0, QROWS, fill, 0)

        def zinit(e, _):
            r0 = sid * ROWS1 + e * QROWS
            pltpu.sync_copy(obuf, acc_a.at[pl.ds(r0, QROWS)])
            pltpu.sync_copy(obuf, acc_b.at[pl.ds(r0, QROWS)])
            pltpu.sync_copy(tab_hbm.at[pl.ds(tab_off + r0, QROWS)],
                            tbl_s.at[pl.ds(r0, QROWS)])
            return 0
        lax.fori_loop(0, 4, zinit, 0)
        plsc.subcore_barrier()

        def run_prop(tbl, acc):
            def quarter(hh, _):
                pltpu.sync_copy(src2_hbm.at[pl.ds(rbase + hh * NCHK_Q, NCHK_Q)], sbuf2)
                pltpu.sync_copy(dst2_hbm.at[pl.ds(rbase + hh * NCHK_Q, NCHK_Q)], dbuf2)

                for t in range(NRING):
                    pltpu.async_copy(tbl.at[sbuf2.at[t]], ring[t], gsem[t])

                def grp(q, _):
                    j0 = q * NRING
                    for t in range(NRING):
                        pltpu.make_async_copy(tab_hbm.at[sbuf2.at[0]], ring[t], gsem[t]).wait()
                        pltpu.async_copy(ring[t], acc.at[dbuf2.at[j0 + t]], ssem[t], add=True)
                    for t in range(NRING):
                        @pl.when(q < NGRP_Q - 1)
                        def _():
                            pltpu.make_async_copy(tab_hbm.at[sbuf2.at[0]], ring[t], ssem[t]).wait()
                            pltpu.async_copy(tbl.at[sbuf2.at[j0 + NRING + t]], ring[t], gsem[t])
                    return 0
                lax.fori_loop(0, NGRP_Q, grp, 0)
                for t in range(NRING):
                    pltpu.make_async_copy(tab_hbm.at[sbuf2.at[0]], ring[t], ssem[t]).wait()
                return 0
            lax.fori_loop(0, 4, quarter, 0)

        def combine(acc, add_h, scal_h, dst_is_spmem):
            def epi(e, _):
                r0 = sid * ROWS1 + e * QROWS
                pltpu.sync_copy(acc.at[pl.ds(r0, QROWS)], abuf)
                pltpu.sync_copy(add_h.at[cid, pl.ds(r0, QROWS)], obuf)
                pltpu.sync_copy(scal_h.at[pl.ds(r0, QROWS)], sbuf)

                def row(i, _):
                    for j in range(FH // 16):
                        s = pl.ds(j * 16, 16)
                        obuf[i, s] = obuf[i, s] + sbuf[i, s] * abuf[i, s]
                    return 0
                lax.fori_loop(0, QROWS, row, 0)
                if dst_is_spmem:
                    pltpu.sync_copy(obuf, acc_a.at[pl.ds(r0, QROWS)])
                else:
                    pltpu.sync_copy(obuf, out_hbm.at[cid, pl.ds(r0, QROWS)])
                return 0
            lax.fori_loop(0, 4, epi, 0)

        run_prop(tbl_s, acc_a)
        plsc.subcore_barrier()
        combine(acc_a, g1_hbm, sa1_hbm, True)
        plsc.subcore_barrier()
        run_prop(acc_a, acc_b)
        plsc.subcore_barrier()
        combine(acc_b, ys_hbm, sb1_hbm, False)

    return k


def _prop_edge_kernel():
    @functools.partial(
        pl.kernel,
        out_type=jax.ShapeDtypeStruct((2, NP, CLS), _f32),
        mesh=_mesh(),
        compiler_params=_SC_PARAMS,
        scratch_types=[
            pltpu.VMEM_SHARED((NP, CLS), _f32),
            pltpu.VMEM((NCHK_E, C), jnp.int32),
            pltpu.VMEM((NCHK_E, C), jnp.int32),
            [pltpu.VMEM((C, CLS), _f32)] * NRING,
            pltpu.VMEM((ROWS1, CLS), _f32),
            [pltpu.SemaphoreType.DMA] * NRING,
            [pltpu.SemaphoreType.DMA] * NRING,
        ],
    )
    def k(src2_hbm, dst2_hbm, tab_hbm, out_hbm,
          acc, sbuf2, dbuf2, ring, rbuf, gsem, ssem):
        cid = lax.axis_index("c")
        sid = lax.axis_index("s")
        zero16 = jnp.zeros((16,), _f32)

        def fill(i, _):
            rbuf[i, :] = zero16
            return 0
        lax.fori_loop(0, ROWS1, fill, 0)
        pltpu.sync_copy(rbuf, acc.at[pl.ds(sid * ROWS1, ROWS1)])
        plsc.subcore_barrier()

        wid = sid * 2 + cid
        rbase = wid * NCHK_E
        pltpu.sync_copy(src2_hbm.at[pl.ds(rbase, NCHK_E)], sbuf2)
        pltpu.sync_copy(dst2_hbm.at[pl.ds(rbase, NCHK_E)], dbuf2)

        for t in range(NRING):
            pltpu.async_copy(tab_hbm.at[sbuf2.at[t]], ring[t], gsem[t])

        NGRP_E = NCHK_E // NRING

        def grp(q, _):
            j0 = q * NRING
            for t in range(NRING):
                pltpu.make_async_copy(tab_hbm.at[sbuf2.at[0]], ring[t], gsem[t]).wait()
                pltpu.async_copy(ring[t], acc.at[dbuf2.at[j0 + t]], ssem[t], add=True)
            for t in range(NRING):
                @pl.when(q < NGRP_E - 1)
                def _():
                    pltpu.make_async_copy(tab_hbm.at[sbuf2.at[0]], ring[t], ssem[t]).wait()
                    pltpu.async_copy(tab_hbm.at[sbuf2.at[j0 + NRING + t]], ring[t], gsem[t])
            return 0
        lax.fori_loop(0, NGRP_E, grp, 0)
        for t in range(NRING):
            pltpu.make_async_copy(tab_hbm.at[sbuf2.at[0]], ring[t], ssem[t]).wait()

        plsc.subcore_barrier()
        r0 = sid * ROWS1
        pltpu.sync_copy(acc.at[pl.ds(r0, ROWS1)], rbuf)
        pltpu.sync_copy(rbuf, out_hbm.at[cid, pl.ds(r0, ROWS1)])

    return k


def _cmb_body(add_ref, scal_ref, p_ref, o_ref):
    o_ref[...] = add_ref[...] + scal_ref[...] * (p_ref[0] + p_ref[1])


def _cmb_call(add, scal, p):
    return pl.pallas_call(
        _cmb_body, out_shape=jax.ShapeDtypeStruct((NP, CLS), _f32))(add, scal, p)


def _fin_body(v_ref, scal_ref, q_ref, b_ref, o_ref):
    res = v_ref[...] + scal_ref[...] * (q_ref[0] + q_ref[1]) + b_ref[...]
    o_ref[...] = res[:N]


def _fin_call(v0m2, sb2, q, b2):
    return pl.pallas_call(
        _fin_body, out_shape=jax.ShapeDtypeStruct((N, CLS), _f32))(
            v0m2, sb2, q, b2.reshape(1, CLS))


def _mm1_body(x_ref, degp_ref, w_ref,
              y_ref, g1_ref, g2_ref, dis_ref, sa1_ref, sb1_ref,
              sa2_ref, sb2_ref):
    deg = degp_ref[0, :, 0:1] + degp_ref[1, :, 0:1]
    dis = jnp.where(deg > 0, jax.lax.rsqrt(deg), 0.0)
    dis_ref[...] = dis
    dis2 = dis * dis
    bm = dis.shape[0]
    sa1_ref[...] = jnp.broadcast_to(-2.0 * dis2, (bm, FH))
    sb1_ref[...] = jnp.broadcast_to(-dis, (bm, FH))
    sa2_ref[...] = jnp.broadcast_to(-2.0 * dis2, (bm, CLS))
    sb2_ref[...] = jnp.broadcast_to(-dis, (bm, CLS))
    x = x_ref[...]
    y = _dot(x, w_ref[0] - w_ref[2])
    g1 = dis * _dot(x, w_ref[1])
    g2 = dis * _dot(x, w_ref[2])
    y_ref[0] = y[:, :FH]
    y_ref[1] = y[:, FH:]
    g1_ref[0] = g1[:, :FH]
    g1_ref[1] = g1[:, FH:]
    g2_ref[0] = g2[:, :FH]
    g2_ref[1] = g2[:, FH:]


def _mm1_call(xp, degp, W1):
    BM = 1024
    grid = (NP // BM,)
    outs = [
        jax.ShapeDtypeStruct((2, NP, FH), _f32),
        jax.ShapeDtypeStruct((2, NP, FH), _f32),
        jax.ShapeDtypeStruct((2, NP, FH), _f32),
        jax.ShapeDtypeStruct((NP, 1), _f32),
        jax.ShapeDtypeStruct((NP, FH), _f32),
        jax.ShapeDtypeStruct((NP, FH), _f32),
        jax.ShapeDtypeStruct((NP, CLS), _f32),
        jax.ShapeDtypeStruct((NP, CLS), _f32),
    ]
    in_specs = [
        pl.BlockSpec((BM, F_IN), lambda i: (i, 0)),
        pl.BlockSpec((2, BM, 16), lambda i: (0, i, 0)),
        pl.BlockSpec((3, F_IN, HID), lambda i: (0, 0, 0)),
    ]
    out_specs = [
        pl.BlockSpec((2, BM, FH), lambda i: (0, i, 0)),
        pl.BlockSpec((2, BM, FH), lambda i: (0, i, 0)),
        pl.BlockSpec((2, BM, FH), lambda i: (0, i, 0)),
        pl.BlockSpec((BM, 1), lambda i: (i, 0)),
        pl.BlockSpec((BM, FH), lambda i: (i, 0)),
        pl.BlockSpec((BM, FH), lambda i: (i, 0)),
        pl.BlockSpec((BM, CLS), lambda i: (i, 0)),
        pl.BlockSpec((BM, CLS), lambda i: (i, 0)),
    ]
    return pl.pallas_call(_mm1_body, grid=grid, in_specs=in_specs,
                          out_specs=out_specs, out_shape=outs)(xp, degp, W1)


def _mm2_body(z_ref, w_ref, b1_ref, dis_ref, v_ref, gg1_ref, gg2_ref):
    z = jnp.concatenate([z_ref[0], z_ref[1]], axis=1)
    h = jnp.maximum(z + b1_ref[...], 0.0)
    dis = dis_ref[...]
    v_ref[...] = _dot(h, w_ref[0] - w_ref[2])
    gg1_ref[...] = dis * _dot(h, w_ref[1])
    gg2_ref[...] = dis * _dot(h, w_ref[2])


def _mm2_call(zs, W2, b1, dis):
    BM = 1024
    grid = (NP // BM,)
    outs = [
        jax.ShapeDtypeStruct((NP, CLS), _f32),
        jax.ShapeDtypeStruct((NP, CLS), _f32),
        jax.ShapeDtypeStruct((NP, CLS), _f32),
    ]
    in_specs = [
        pl.BlockSpec((2, BM, FH), lambda i: (0, i, 0)),
        pl.BlockSpec((3, HID, CLS), lambda i: (0, 0, 0)),
        pl.BlockSpec((1, HID), lambda i: (0, 0)),
        pl.BlockSpec((BM, 1), lambda i: (i, 0)),
    ]
    out_specs = [
        pl.BlockSpec((BM, CLS), lambda i: (i, 0)),
        pl.BlockSpec((BM, CLS), lambda i: (i, 0)),
        pl.BlockSpec((BM, CLS), lambda i: (i, 0)),
    ]
    return pl.pallas_call(_mm2_body, grid=grid, in_specs=in_specs,
                          out_specs=out_specs,
                          out_shape=outs)(zs, W2, b1.reshape(1, HID), dis)


_K_DEG = _deg_kernel()
_K_L1 = _layer1_kernel()
_K_E = _prop_edge_kernel()


def kernel(x, edge_index, W1, b1, W2, b2):
    xp = jnp.pad(x, ((0, NP - N), (0, 0)))
    pad = EPAD - E
    padv = jnp.full((pad,), NP - 1, jnp.int32)
    src2 = jnp.concatenate([edge_index[0], padv]).reshape(EPAD // C, C)
    dst2 = jnp.concatenate([edge_index[1], padv]).reshape(EPAD // C, C)

    degp = _K_DEG(src2)
    ys, g1s, g2s, dis, sa1, sb1, sa2, sb2 = _mm1_call(xp, degp, W1)

    g2f = g2s.reshape(2 * NP, FH)
    zs = _K_L1(src2, dst2, g2f, g1s, ys, sa1, sb1)

    v0m2, gg1, gg2 = _mm2_call(zs, W2, b1, dis)
    p = _K_E(src2, dst2, gg2)
    gp2 = _cmb_call(gg1, sa2, p)
    q = _K_E(src2, dst2, gp2)
    return _fin_call(v0m2, sb2, q, b2)


# speedup vs baseline: 22.6725x; 1.1212x over previous
import functools

import jax
import jax.numpy as jnp
from jax import lax
from jax.experimental import pallas as pl
from jax.experimental.pallas import tpu as pltpu
from jax.experimental.pallas import tpu_sc as plsc

N = 10000
E = 320000
NP = 10240
F_IN = 128
HID = 64
CLS = 16
FH = HID // 2
HALFN = NP // 2
ACCR = HALFN + 8
DUMMY = HALFN
C = 128
EPAD = 327680
NSUB = 16
ROWS1 = NP // NSUB
ROWS2 = HALFN // NSUB
NCHK_E = EPAD // 32 // C
NCHK_C = EPAD // 16 // C
NB_D = 16
NRING = 8
NCHK_H = NCHK_C // 2
NGRP = NCHK_H // NRING
NCHK_Q = NCHK_C // 4
NGRP_Q = NCHK_Q // NRING
EROWS = ROWS1 // 2
QROWS = ROWS1 // 4

_mesh = lambda: plsc.VectorSubcoreMesh(core_axis_name="c", subcore_axis_name="s")
_SC_PARAMS = pltpu.CompilerParams(use_tc_tiling_on_sc=False)
_f32 = jnp.float32
_HIGH = jax.lax.Precision.HIGHEST


def _dot(a, b):
    return jax.lax.dot(a, b, precision=_HIGH, preferred_element_type=_f32)


def _deg_kernel():
    @functools.partial(
        pl.kernel,
        out_type=jax.ShapeDtypeStruct((2, NP, 16), _f32),
        mesh=_mesh(),
        compiler_params=_SC_PARAMS,
        scratch_types=[
            pltpu.VMEM_SHARED((NP, 16), _f32),
            pltpu.VMEM((NB_D, C), jnp.int32),
            pltpu.VMEM((C, 16), _f32),
            pltpu.VMEM((ROWS1, 16), _f32),
            pltpu.SemaphoreType.DMA,
        ],
    )
    def k(src2_hbm, out_hbm, acc, sbuf2, ones_v, rbuf, ssem):
        cid = lax.axis_index("c")
        sid = lax.axis_index("s")
        zero16 = jnp.zeros((16,), _f32)
        pat = jnp.where(lax.iota(jnp.int32, 16) == 0, 1.0, 0.0).astype(_f32)

        def fill(i, _):
            rbuf[i, :] = zero16
            return 0
        lax.fori_loop(0, ROWS1, fill, 0)

        def fill1(i, _):
            ones_v[i, :] = pat
            return 0
        lax.fori_loop(0, C, fill1, 0)

        pltpu.sync_copy(rbuf, acc.at[pl.ds(sid * ROWS1, ROWS1)])
        plsc.subcore_barrier()

        wid = sid * 2 + cid
        rbase = wid * NCHK_E

        def blk(b, _):
            pltpu.sync_copy(src2_hbm.at[pl.ds(rbase + b * NB_D, NB_D)], sbuf2)

            def fire(j, _):
                pltpu.async_copy(ones_v, acc.at[sbuf2.at[j]], ssem, add=True)
                return 0
            lax.fori_loop(0, NB_D, fire, 0)

            def drain(j, _):
                pltpu.make_async_copy(out_hbm.at[0, pl.ds(0, C)], ones_v, ssem).wait()
                return 0
            lax.fori_loop(0, NB_D, drain, 0)
            return 0
        lax.fori_loop(0, NCHK_E // NB_D, blk, 0)

        plsc.subcore_barrier()
        r0 = sid * ROWS1
        pltpu.sync_copy(acc.at[pl.ds(r0, ROWS1)], rbuf)
        pltpu.sync_copy(rbuf, out_hbm.at[cid, pl.ds(r0, ROWS1)])

    return k


def _layer1_kernel():
    @functools.partial(
        pl.kernel,
        out_type=jax.ShapeDtypeStruct((2, NP, FH), _f32),
        mesh=_mesh(),
        compiler_params=_SC_PARAMS,
        scratch_types=[
            pltpu.VMEM_SHARED((NP, FH), _f32),
            pltpu.VMEM_SHARED((NP, FH), _f32),
            pltpu.VMEM_SHARED((NP, FH), _f32),
            pltpu.VMEM((NCHK_Q, C), jnp.int32),
            pltpu.VMEM((NCHK_Q, C), jnp.int32),
            [pltpu.VMEM((C, FH), _f32)] * NRING,
            pltpu.VMEM((QROWS, FH), _f32),
            pltpu.VMEM((QROWS, FH), _f32),
            pltpu.VMEM((QROWS, FH), _f32),
            [pltpu.SemaphoreType.DMA] * NRING,
            [pltpu.SemaphoreType.DMA] * NRING,
        ],
    )
    def k(src2_hbm, dst2_hbm, tab_hbm, g1_hbm, ys_hbm, sa1_hbm, sb1_hbm, out_hbm,
          tbl_s, acc_a, acc_b, sbuf2, dbuf2, ring, abuf, obuf, sbuf, gsem, ssem):
        cid = lax.axis_index("c")
        sid = lax.axis_index("s")
        zero16 = jnp.zeros((16,), _f32)
        rbase = sid * NCHK_C
        tab_off = cid * NP

        def fill(i, _):
            obuf[i, pl.ds(0, 16)] = zero16
            obuf[i, pl.ds(16, 16)] = zero16
            return 0
        lax.fori_loop(0, QROWS, fill, 0)

        def zinit(e, _):
            r0 = sid * ROWS1 + e * QROWS
            pltpu.sync_copy(obuf, acc_a.at[pl.ds(r0, QROWS)])
            pltpu.sync_copy(obuf, acc_b.at[pl.ds(r0, QROWS)])
            pltpu.sync_copy(tab_hbm.at[pl.ds(tab_off + r0, QROWS)],
                            tbl_s.at[pl.ds(r0, QROWS)])
            return 0
        lax.fori_loop(0, 4, zinit, 0)
        plsc.subcore_barrier()

        def run_prop(tbl, acc):
            def quarter(hh, _):
                pltpu.sync_copy(src2_hbm.at[pl.ds(rbase + hh * NCHK_Q, NCHK_Q)], sbuf2)
                pltpu.sync_copy(dst2_hbm.at[pl.ds(rbase + hh * NCHK_Q, NCHK_Q)], dbuf2)

                for t in range(NRING):
                    pltpu.async_copy(tbl.at[sbuf2.at[t]], ring[t], gsem[t])

                def grp(q, _):
                    j0 = q * NRING
                    for t in range(NRING):
                        pltpu.make_async_copy(tab_hbm.at[sbuf2.at[0]], ring[t], gsem[t]).wait()
                        pltpu.async_copy(ring[t], acc.at[dbuf2.at[j0 + t]], ssem[t], add=True)
                    for t in range(NRING):
                        @pl.when(q < NGRP_Q - 1)
                        def _():
                            pltpu.make_async_copy(tab_hbm.at[sbuf2.at[0]], ring[t], ssem[t]).wait()
                            pltpu.async_copy(tbl.at[sbuf2.at[j0 + NRING + t]], ring[t], gsem[t])
                    return 0
                lax.fori_loop(0, NGRP_Q, grp, 0)
                for t in range(NRING):
                    pltpu.make_async_copy(tab_hbm.at[sbuf2.at[0]], ring[t], ssem[t]).wait()
                return 0
            lax.fori_loop(0, 4, quarter, 0)

        def combine(acc, add_h, scal_h, dst_is_spmem):
            def epi(e, _):
                r0 = sid * ROWS1 + e * QROWS
                pltpu.sync_copy(acc.at[pl.ds(r0, QROWS)], abuf)
                pltpu.sync_copy(add_h.at[cid, pl.ds(r0, QROWS)], obuf)
                pltpu.sync_copy(scal_h.at[pl.ds(r0, QROWS)], sbuf)

                def row(i, _):
                    for j in range(FH // 16):
                        s = pl.ds(j * 16, 16)
                        obuf[i, s] = obuf[i, s] + sbuf[i, s] * abuf[i, s]
                    return 0
                lax.fori_loop(0, QROWS, row, 0)
                if dst_is_spmem:
                    pltpu.sync_copy(obuf, acc_a.at[pl.ds(r0, QROWS)])
                else:
                    pltpu.sync_copy(obuf, out_hbm.at[cid, pl.ds(r0, QROWS)])
                return 0
            lax.fori_loop(0, 4, epi, 0)

        run_prop(tbl_s, acc_a)
        plsc.subcore_barrier()
        combine(acc_a, g1_hbm, sa1_hbm, True)
        plsc.subcore_barrier()
        run_prop(acc_a, acc_b)
        plsc.subcore_barrier()
        combine(acc_b, ys_hbm, sb1_hbm, False)

    return k


def _prop_edge_kernel():
    @functools.partial(
        pl.kernel,
        out_type=jax.ShapeDtypeStruct((2, NP, CLS), _f32),
        mesh=_mesh(),
        compiler_params=_SC_PARAMS,
        scratch_types=[
            pltpu.VMEM_SHARED((NP, CLS), _f32),
            pltpu.VMEM_SHARED((NP, CLS), _f32),
            pltpu.VMEM((NCHK_E, C), jnp.int32),
            pltpu.VMEM((NCHK_E, C), jnp.int32),
            [pltpu.VMEM((C, CLS), _f32)] * NRING,
            pltpu.VMEM((ROWS1, CLS), _f32),
            [pltpu.SemaphoreType.DMA] * NRING,
            [pltpu.SemaphoreType.DMA] * NRING,
        ],
    )
    def k(src2_hbm, dst2_hbm, tab_hbm, out_hbm,
          acc, tbl_s, sbuf2, dbuf2, ring, rbuf, gsem, ssem):
        cid = lax.axis_index("c")
        sid = lax.axis_index("s")
        zero16 = jnp.zeros((16,), _f32)

        def fill(i, _):
            rbuf[i, :] = zero16
            return 0
        lax.fori_loop(0, ROWS1, fill, 0)
        pltpu.sync_copy(rbuf, acc.at[pl.ds(sid * ROWS1, ROWS1)])
        pltpu.sync_copy(tab_hbm.at[pl.ds(sid * ROWS1, ROWS1)],
                        tbl_s.at[pl.ds(sid * ROWS1, ROWS1)])
        plsc.subcore_barrier()

        wid = sid * 2 + cid
        rbase = wid * NCHK_E
        pltpu.sync_copy(src2_hbm.at[pl.ds(rbase, NCHK_E)], sbuf2)
        pltpu.sync_copy(dst2_hbm.at[pl.ds(rbase, NCHK_E)], dbuf2)

        for t in range(NRING):
            pltpu.async_copy(tbl_s.at[sbuf2.at[t]], ring[t], gsem[t])

        NGRP_E = NCHK_E // NRING

        def grp(q, _):
            j0 = q * NRING
            for t in range(NRING):
                pltpu.make_async_copy(tab_hbm.at[sbuf2.at[0]], ring[t], gsem[t]).wait()
                pltpu.async_copy(ring[t], acc.at[dbuf2.at[j0 + t]], ssem[t], add=True)
            for t in range(NRING):
                @pl.when(q < NGRP_E - 1)
                def _():
                    pltpu.make_async_copy(tab_hbm.at[sbuf2.at[0]], ring[t], ssem[t]).wait()
                    pltpu.async_copy(tbl_s.at[sbuf2.at[j0 + NRING + t]], ring[t], gsem[t])
            return 0
        lax.fori_loop(0, NGRP_E, grp, 0)
        for t in range(NRING):
            pltpu.make_async_copy(tab_hbm.at[sbuf2.at[0]], ring[t], ssem[t]).wait()

        plsc.subcore_barrier()
        r0 = sid * ROWS1
        pltpu.sync_copy(acc.at[pl.ds(r0, ROWS1)], rbuf)
        pltpu.sync_copy(rbuf, out_hbm.at[cid, pl.ds(r0, ROWS1)])

    return k


def _cmb_body(add_ref, scal_ref, p_ref, o_ref):
    o_ref[...] = add_ref[...] + scal_ref[...] * (p_ref[0] + p_ref[1])


def _cmb_call(add, scal, p):
    return pl.pallas_call(
        _cmb_body, out_shape=jax.ShapeDtypeStruct((NP, CLS), _f32))(add, scal, p)


def _fin_body(v_ref, scal_ref, q_ref, b_ref, o_ref):
    res = v_ref[...] + scal_ref[...] * (q_ref[0] + q_ref[1]) + b_ref[...]
    o_ref[...] = res[:N]


def _fin_call(v0m2, sb2, q, b2):
    return pl.pallas_call(
        _fin_body, out_shape=jax.ShapeDtypeStruct((N, CLS), _f32))(
            v0m2, sb2, q, b2.reshape(1, CLS))


def _mm1_body(x_ref, degp_ref, w_ref,
              y_ref, g1_ref, g2_ref, dis_ref, sa1_ref, sb1_ref,
              sa2_ref, sb2_ref):
    deg = degp_ref[0, :, 0:1] + degp_ref[1, :, 0:1]
    dis = jnp.where(deg > 0, jax.lax.rsqrt(deg), 0.0)
    dis_ref[...] = dis
    dis2 = dis * dis
    bm = dis.shape[0]
    sa1_ref[...] = jnp.broadcast_to(-2.0 * dis2, (bm, FH))
    sb1_ref[...] = jnp.broadcast_to(-dis, (bm, FH))
    sa2_ref[...] = jnp.broadcast_to(-2.0 * dis2, (bm, CLS))
    sb2_ref[...] = jnp.broadcast_to(-dis, (bm, CLS))
    x = x_ref[...]
    y = _dot(x, w_ref[0] - w_ref[2])
    g1 = dis * _dot(x, w_ref[1])
    g2 = dis * _dot(x, w_ref[2])
    y_ref[0] = y[:, :FH]
    y_ref[1] = y[:, FH:]
    g1_ref[0] = g1[:, :FH]
    g1_ref[1] = g1[:, FH:]
    g2_ref[0] = g2[:, :FH]
    g2_ref[1] = g2[:, FH:]


def _mm1_call(xp, degp, W1):
    BM = 1024
    grid = (NP // BM,)
    outs = [
        jax.ShapeDtypeStruct((2, NP, FH), _f32),
        jax.ShapeDtypeStruct((2, NP, FH), _f32),
        jax.ShapeDtypeStruct((2, NP, FH), _f32),
        jax.ShapeDtypeStruct((NP, 1), _f32),
        jax.ShapeDtypeStruct((NP, FH), _f32),
        jax.ShapeDtypeStruct((NP, FH), _f32),
        jax.ShapeDtypeStruct((NP, CLS), _f32),
        jax.ShapeDtypeStruct((NP, CLS), _f32),
    ]
    in_specs = [
        pl.BlockSpec((BM, F_IN), lambda i: (i, 0)),
        pl.BlockSpec((2, BM, 16), lambda i: (0, i, 0)),
        pl.BlockSpec((3, F_IN, HID), lambda i: (0, 0, 0)),
    ]
    out_specs = [
        pl.BlockSpec((2, BM, FH), lambda i: (0, i, 0)),
        pl.BlockSpec((2, BM, FH), lambda i: (0, i, 0)),
        pl.BlockSpec((2, BM, FH), lambda i: (0, i, 0)),
        pl.BlockSpec((BM, 1), lambda i: (i, 0)),
        pl.BlockSpec((BM, FH), lambda i: (i, 0)),
        pl.BlockSpec((BM, FH), lambda i: (i, 0)),
        pl.BlockSpec((BM, CLS), lambda i: (i, 0)),
        pl.BlockSpec((BM, CLS), lambda i: (i, 0)),
    ]
    return pl.pallas_call(_mm1_body, grid=grid, in_specs=in_specs,
                          out_specs=out_specs, out_shape=outs)(xp, degp, W1)


def _mm2_body(z_ref, w_ref, b1_ref, dis_ref, v_ref, gg1_ref, gg2_ref):
    z = jnp.concatenate([z_ref[0], z_ref[1]], axis=1)
    h = jnp.maximum(z + b1_ref[...], 0.0)
    dis = dis_ref[...]
    v_ref[...] = _dot(h, w_ref[0] - w_ref[2])
    gg1_ref[...] = dis * _dot(h, w_ref[1])
    gg2_ref[...] = dis * _dot(h, w_ref[2])


def _mm2_call(zs, W2, b1, dis):
    BM = 1024
    grid = (NP // BM,)
    outs = [
        jax.ShapeDtypeStruct((NP, CLS), _f32),
        jax.ShapeDtypeStruct((NP, CLS), _f32),
        jax.ShapeDtypeStruct((NP, CLS), _f32),
    ]
    in_specs = [
        pl.BlockSpec((2, BM, FH), lambda i: (0, i, 0)),
        pl.BlockSpec((3, HID, CLS), lambda i: (0, 0, 0)),
        pl.BlockSpec((1, HID), lambda i: (0, 0)),
        pl.BlockSpec((BM, 1), lambda i: (i, 0)),
    ]
    out_specs = [
        pl.BlockSpec((BM, CLS), lambda i: (i, 0)),
        pl.BlockSpec((BM, CLS), lambda i: (i, 0)),
        pl.BlockSpec((BM, CLS), lambda i: (i, 0)),
    ]
    return pl.pallas_call(_mm2_body, grid=grid, in_specs=in_specs,
                          out_specs=out_specs,
                          out_shape=outs)(zs, W2, b1.reshape(1, HID), dis)


_K_DEG = _deg_kernel()
_K_L1 = _layer1_kernel()
_K_E = _prop_edge_kernel()


def kernel(x, edge_index, W1, b1, W2, b2):
    xp = jnp.pad(x, ((0, NP - N), (0, 0)))
    pad = EPAD - E
    padv = jnp.full((pad,), NP - 1, jnp.int32)
    src2 = jnp.concatenate([edge_index[0], padv]).reshape(EPAD // C, C)
    dst2 = jnp.concatenate([edge_index[1], padv]).reshape(EPAD // C, C)

    degp = _K_DEG(src2)
    ys, g1s, g2s, dis, sa1, sb1, sa2, sb2 = _mm1_call(xp, degp, W1)

    g2f = g2s.reshape(2 * NP, FH)
    zs = _K_L1(src2, dst2, g2f, g1s, ys, sa1, sb1)

    v0m2, gg1, gg2 = _mm2_call(zs, W2, b1, dis)
    p = _K_E(src2, dst2, gg2)
    gp2 = _cmb_call(gg1, sa2, p)
    q = _K_E(src2, dst2, gp2)
    return _fin_call(v0m2, sb2, q, b2)
